# baseline (speedup 1.0000x reference)
import functools

import jax
import jax.numpy as jnp
from jax import lax
from jax.experimental import pallas as pl
from jax.experimental.pallas import tpu as pltpu


def _gcn_body(N, L, C, P, nft_ref, tt_ref, m_ref, b_ref, o_ref):
    bt = nft_ref.shape[-1]

    mm = jnp.dot(m_ref[...], nft_ref[...],
                 preferred_element_type=jnp.float32)
    xw = mm[0:N * C].reshape(N, C, bt)
    idx_f = mm[N * C:N * C + N]
    lead = mm[N * C + N:N * C + 2 * N]
    idx = jnp.where(lead >= 0.0, idx_f, -1.0).astype(jnp.int32)
    idx3 = idx.reshape(N, 1, bt)

    tbit = tt_ref[...].astype(jnp.int32).reshape(L, L, bt)
    pow2 = jnp.left_shift(
        jnp.int32(1), lax.broadcasted_iota(jnp.int32, (1, L, 1), 1))
    packed = jnp.sum(tbit * pow2, axis=1).reshape(L, 1, bt)

    prow = jnp.where(idx3 == 0, packed[0:1], 0)
    for l1 in range(1, L):
        prow = prow + jnp.where(idx3 == l1, packed[l1:l1 + 1], 0)

    shamt = jnp.where(idx3 >= 0, idx3, 31).reshape(1, N, bt)
    adj = ((prow >> shamt) & 1).astype(jnp.float32)
    eye = (lax.broadcasted_iota(jnp.int32, (N, N, bt), 0) ==
           lax.broadcasted_iota(jnp.int32, (N, N, bt), 1)).astype(jnp.float32)
    adj = jnp.maximum(adj, eye)
    deg = jnp.sum(adj, axis=1, keepdims=True)
    dinv = lax.rsqrt(deg)

    xw_s = xw * dinv
    acc = adj[:, 0:1] * xw_s[0:1]
    for j in range(1, N):
        acc = acc + adj[:, j:j + 1] * xw_s[j:j + 1]
    out = (acc * dinv).reshape(N * C, bt).T
    o_ref[...] = out + b_ref[...]


def _round_up(x, m):
    return ((x + m - 1) // m) * m


@functools.partial(jax.jit, static_argnames=("block_b",))
def _gcn_forward(node_features, topo_outputs, w, b, *, block_b=1024):
    node_features = jnp.asarray(node_features, jnp.float32)
    topo_outputs = jnp.asarray(topo_outputs, jnp.float32)
    w = jnp.asarray(w, jnp.float32)
    b = jnp.asarray(b, jnp.float32)

    B, N, P = node_features.shape
    L = topo_outputs.shape[-1]
    C = w.shape[-1]

    bt = max(128, min(int(block_b), _round_up(B, 128)))
    b_pad = _round_up(B, bt)
    pad = b_pad - B
    if pad:
        node_features = jnp.concatenate(
            [node_features, jnp.full((pad, N, P), -1.0, jnp.float32)], axis=0)
        topo_outputs = jnp.concatenate(
            [topo_outputs, jnp.zeros((pad, L, L), jnp.float32)], axis=0)

    nft = jnp.transpose(node_features, (1, 2, 0)).reshape(N * P, b_pad)
    t8 = (topo_outputs >= 0.0).astype(jnp.int8)
    tt = jnp.transpose(t8, (1, 2, 0)).reshape(L * L, b_pad)

    eyeN = jnp.eye(N, dtype=jnp.float32)
    pw = jnp.asarray([2.0 ** (P - 2 - k) for k in range(P - 1)] + [0.0],
                     jnp.float32)[None, :]
    e0 = jnp.zeros((1, P), jnp.float32).at[0, 0].set(1.0)
    m_mat = jnp.concatenate(
        [jnp.kron(eyeN, w.T), jnp.kron(eyeN, pw), jnp.kron(eyeN, e0)], axis=0)
    b_rep = jnp.tile(b, (N,))[None, :]

    out2 = pl.pallas_call(
        functools.partial(_gcn_body, N, L, C, P),
        out_shape=jax.ShapeDtypeStruct((b_pad, N * C), jnp.float32),
        grid=(b_pad // bt,),
        in_specs=[
            pl.BlockSpec((N * P, bt), lambda g: (0, g)),
            pl.BlockSpec((L * L, bt), lambda g: (0, g)),
            pl.BlockSpec((N * C + 2 * N, N * P), lambda g: (0, 0)),
            pl.BlockSpec((1, N * C), lambda g: (0, 0)),
        ],
        out_specs=pl.BlockSpec((bt, N * C), lambda g: (g, 0)),
        compiler_params=pltpu.CompilerParams(
            dimension_semantics=("parallel",)),
    )(nft, tt, m_mat, b_rep)

    return out2[:B].reshape(B, N, C)


def kernel(node_features, topo_outputs, w, b):
    return _gcn_forward(node_features, topo_outputs, w, b)


# speedup vs baseline: 1.3597x; 1.3597x over previous
import functools

import jax
import jax.numpy as jnp
from jax import lax
from jax.experimental import pallas as pl
from jax.experimental.pallas import tpu as pltpu


def _gcn_body(N, L, C, P, nft_ref, tt_ref, m_ref, b_ref, o_ref):
    bt = nft_ref.shape[-1]

    mm = jnp.dot(m_ref[...], nft_ref[...],
                 preferred_element_type=jnp.float32)
    xw = mm[0:N * C].reshape(N, C, bt)
    idx_f = mm[N * C:N * C + N]
    lead = mm[N * C + N:N * C + 2 * N]
    idx = jnp.where(lead >= 0.0, idx_f, -1.0).astype(jnp.int32)
    idx3 = idx.reshape(N, 1, bt)

    tbit = (tt_ref[...] >= 0.0).astype(jnp.int32).reshape(L, L, bt)
    pow2 = jnp.left_shift(
        jnp.int32(1), lax.broadcasted_iota(jnp.int32, (1, L, 1), 1))
    packed = jnp.sum(tbit * pow2, axis=1).reshape(L, 1, bt)

    prow = jnp.where(idx3 == 0, packed[0:1], 0)
    for l1 in range(1, L):
        prow = prow + jnp.where(idx3 == l1, packed[l1:l1 + 1], 0)

    shamt = jnp.where(idx3 >= 0, idx3, 31).reshape(1, N, bt)
    adj = ((prow >> shamt) & 1).astype(jnp.float32)
    eye = (lax.broadcasted_iota(jnp.int32, (N, N, bt), 0) ==
           lax.broadcasted_iota(jnp.int32, (N, N, bt), 1)).astype(jnp.float32)
    adj = jnp.maximum(adj, eye)
    deg = jnp.sum(adj, axis=1, keepdims=True)
    dinv = lax.rsqrt(deg)

    xw_s = xw * dinv
    acc = adj[:, 0:1] * xw_s[0:1]
    for j in range(1, N):
        acc = acc + adj[:, j:j + 1] * xw_s[j:j + 1]
    out = (acc * dinv).reshape(N * C, bt).T
    o_ref[...] = out + b_ref[...]


def _round_up(x, m):
    return ((x + m - 1) // m) * m


@functools.partial(jax.jit, static_argnames=("block_b",))
def _gcn_forward(node_features, topo_outputs, w, b, *, block_b=2048):
    node_features = jnp.asarray(node_features, jnp.float32)
    topo_outputs = jnp.asarray(topo_outputs, jnp.float32)
    w = jnp.asarray(w, jnp.float32)
    b = jnp.asarray(b, jnp.float32)

    B, N, P = node_features.shape
    L = topo_outputs.shape[-1]
    C = w.shape[-1]

    bt = max(128, min(int(block_b), _round_up(B, 128)))
    b_pad = _round_up(B, bt)
    pad = b_pad - B
    if pad:
        node_features = jnp.concatenate(
            [node_features, jnp.full((pad, N, P), -1.0, jnp.float32)], axis=0)
        topo_outputs = jnp.concatenate(
            [topo_outputs, jnp.zeros((pad, L, L), jnp.float32)], axis=0)

    nft = jnp.transpose(node_features, (1, 2, 0)).reshape(N * P, b_pad)
    tt = jnp.transpose(topo_outputs, (1, 2, 0)).reshape(L * L, b_pad)

    eyeN = jnp.eye(N, dtype=jnp.float32)
    pw = jnp.asarray([2.0 ** (P - 2 - k) for k in range(P - 1)] + [0.0],
                     jnp.float32)[None, :]
    e0 = jnp.zeros((1, P), jnp.float32).at[0, 0].set(1.0)
    m_mat = jnp.concatenate(
        [jnp.kron(eyeN, w.T), jnp.kron(eyeN, pw), jnp.kron(eyeN, e0)], axis=0)
    b_rep = jnp.tile(b, (N,))[None, :]

    out2 = pl.pallas_call(
        functools.partial(_gcn_body, N, L, C, P),
        out_shape=jax.ShapeDtypeStruct((b_pad, N * C), jnp.float32),
        grid=(b_pad // bt,),
        in_specs=[
            pl.BlockSpec((N * P, bt), lambda g: (0, g)),
            pl.BlockSpec((L * L, bt), lambda g: (0, g)),
            pl.BlockSpec((N * C + 2 * N, N * P), lambda g: (0, 0)),
            pl.BlockSpec((1, N * C), lambda g: (0, 0)),
        ],
        out_specs=pl.BlockSpec((bt, N * C), lambda g: (g, 0)),
        compiler_params=pltpu.CompilerParams(
            dimension_semantics=("parallel",)),
    )(nft, tt, m_mat, b_rep)

    return out2[:B].reshape(B, N, C)


def kernel(node_features, topo_outputs, w, b):
    return _gcn_forward(node_features, topo_outputs, w, b)
